# baseline (speedup 1.0000x reference)
import functools

import jax
import jax.numpy as jnp
from jax import lax
from jax.experimental import pallas as pl
from jax.experimental.pallas import tpu as pltpu
from jax.experimental.pallas import tpu_sc as plsc

N = 10000
D = 128
E = 320000

NC = 2
NS = 16
NW = NC * NS
EPW = E // NW
CH = 96
GC = 6
NG = 18
EPWP = NG * GC * CH
NACC = 10016
RPT = 624
TAIL0 = RPT * NS
TAILN = N - TAIL0
ZTAILN = NACC - TAIL0

_sc_mesh = plsc.VectorSubcoreMesh(core_axis_name="c", subcore_axis_name="s")


@functools.partial(
    pl.kernel,
    mesh=_sc_mesh,
    out_type=jax.ShapeDtypeStruct((NC, N, D), jnp.float32),
    scratch_types=[
        pltpu.VMEM((GC, CH), jnp.int32),
        pltpu.VMEM((GC, CH), jnp.int32),
        pltpu.VMEM((GC, CH), jnp.int32),
        pltpu.VMEM((GC, CH), jnp.int32),
        pltpu.VMEM((CH, D), jnp.float32),
        pltpu.VMEM((CH, D), jnp.float32),
        pltpu.VMEM((CH, D), jnp.float32),
        pltpu.VMEM_SHARED((NACC, D), jnp.float32),
        pltpu.SemaphoreType.DMA,
        pltpu.SemaphoreType.DMA,
        pltpu.SemaphoreType.DMA,
        pltpu.SemaphoreType.DMA,
    ],
)
def _segsum_sc(m_hbm, src_hbm, dst_hbm, z_hbm, out_hbm,
               si0, si1, di0, di1, r0buf, r1buf, r2buf, acc,
               sem_a, sem_b, sem_c, sem_i):
    cid = lax.axis_index("c")
    sid = lax.axis_index("s")
    wid = cid * NS + sid

    row0 = sid * RPT
    pltpu.sync_copy(z_hbm.at[pl.ds(row0, RPT)], acc.at[pl.ds(row0, RPT)])

    @pl.when(sid == NS - 1)
    def _():
        pltpu.sync_copy(z_hbm.at[pl.ds(TAIL0, ZTAILN)],
                        acc.at[pl.ds(TAIL0, ZTAILN)])

    def idx_load_one(arr, g, ref):
        pltpu.async_copy(arr.at[wid, g], ref, sem_i)

    def idx_wait_one(arr, ref):
        pltpu.make_async_copy(arr.at[wid, 0], ref, sem_i).wait()

    def gather(sref, k, rbuf, sem):
        pltpu.async_copy(m_hbm.at[sref.at[k]], rbuf, sem)

    def gather_wait(rbuf, sem):
        pltpu.make_async_copy(m_hbm.at[si0.at[0]], rbuf, sem).wait()

    def scatter(rbuf, dref, k):
        pltpu.sync_copy(rbuf, acc.at[dref.at[k]], add=True)

    bufs = [(r0buf, sem_a), (r1buf, sem_b), (r2buf, sem_c)]

    pltpu.sync_copy(src_hbm.at[wid, 0], si0)
    pltpu.sync_copy(dst_hbm.at[wid, 0], di0)
    plsc.subcore_barrier()
    idx_load_one(src_hbm, 1, si1)
    idx_load_one(dst_hbm, 1, di1)
    for j in range(3):
        gather(si0, j, bufs[j][0], bufs[j][1])

    def body(g2, _):
        for j in range(6):
            rbuf, sem = bufs[j % 3]
            gather_wait(rbuf, sem)
            if j == 0:
                @pl.when(g2 > 0)
                def _():
                    idx_wait_one(dst_hbm, di0)
            if j == 3:
                idx_wait_one(src_hbm, si1)
            scatter(rbuf, di0, j)
            if j < 3:
                gather(si0, j + 3, rbuf, sem)
            else:
                gather(si1, j - 3, rbuf, sem)
        for j in range(6):
            rbuf, sem = bufs[j % 3]
            gather_wait(rbuf, sem)
            if j == 0:
                idx_wait_one(dst_hbm, di1)
            scatter(rbuf, di1, j)
            if j == 0:
                @pl.when(g2 < NG // 2 - 1)
                def _():
                    idx_load_one(src_hbm, 2 * g2 + 2, si0)
            if j == 1:
                @pl.when(g2 < NG // 2 - 1)
                def _():
                    idx_load_one(dst_hbm, 2 * g2 + 2, di0)
            if j < 3:
                gather(si1, j + 3, rbuf, sem)
            else:
                @pl.when(g2 < NG // 2 - 1)
                def _(j=j, rbuf=rbuf, sem=sem):
                    if j == 3:
                        idx_wait_one(src_hbm, si0)
                    gather(si0, j - 3, rbuf, sem)

        @pl.when(g2 < NG // 2 - 1)
        def _():
            idx_load_one(src_hbm, 2 * g2 + 3, si1)
            idx_load_one(dst_hbm, 2 * g2 + 3, di1)

        return 0

    lax.fori_loop(0, NG // 2, body, 0)

    plsc.subcore_barrier()
    pltpu.sync_copy(acc.at[pl.ds(row0, RPT)],
                    out_hbm.at[cid, pl.ds(row0, RPT)])

    @pl.when(sid == NS - 1)
    def _():
        pltpu.sync_copy(acc.at[pl.ds(TAIL0, TAILN)],
                        out_hbm.at[cid, pl.ds(TAIL0, TAILN)])


_BL = 1000


def _mm(pa, pb, b, W, relu):
    def body(pa_ref, pb_ref, b_ref, w_ref, o_ref):
        s = pa_ref[...] + pb_ref[...]
        y = jnp.dot(s, w_ref[...], preferred_element_type=jnp.float32)
        y = y + b_ref[...]
        o_ref[...] = jnp.maximum(y, 0.0) if relu else y

    return pl.pallas_call(
        body,
        grid=(N // _BL,),
        in_specs=[pl.BlockSpec((_BL, D), lambda i: (i, 0)),
                  pl.BlockSpec((_BL, D), lambda i: (i, 0)),
                  pl.BlockSpec((1, D), lambda i: (0, 0)),
                  pl.BlockSpec((D, D), lambda i: (0, 0))],
        out_specs=pl.BlockSpec((_BL, D), lambda i: (i, 0)),
        out_shape=jax.ShapeDtypeStruct((N, D), jnp.float32),
    )(pa, pb, b, W)


@jax.jit
def kernel(x, adj, W0, b0, W1, b1, W2, b2):
    npad = EPWP - EPW
    pad_src = jnp.broadcast_to(
        (jnp.arange(npad, dtype=jnp.int32) % 64)[None, :], (NW, npad))
    pad_dst = jnp.broadcast_to(
        (N + jnp.arange(npad, dtype=jnp.int32) % (NACC - N))[None, :],
        (NW, npad))
    src2 = jnp.concatenate(
        [adj[0].reshape(NW, EPW), pad_src], axis=1).reshape(NW, NG, GC, CH)
    dst2 = jnp.concatenate(
        [adj[1].reshape(NW, EPW), pad_dst], axis=1).reshape(NW, NG, GC, CH)
    zeros = jnp.zeros((NACC, D), jnp.float32)
    b0r = b0.reshape(1, D)
    b1r = b1.reshape(1, D)
    b2r = b2.reshape(1, D)

    p = _segsum_sc(x, src2, dst2, zeros)
    h = _mm(p[0], p[1], b0r, W0, relu=True)
    p = _segsum_sc(h, src2, dst2, zeros)
    h = _mm(p[0], p[1], b1r, W1, relu=True)
    p = _segsum_sc(h, src2, dst2, zeros)
    return _mm(p[0], p[1], b2r, W2, relu=False)


# speedup vs baseline: 1.0422x; 1.0422x over previous
import functools

import jax
import jax.numpy as jnp
from jax import lax
from jax.experimental import pallas as pl
from jax.experimental.pallas import tpu as pltpu
from jax.experimental.pallas import tpu_sc as plsc

N = 10000
D = 128
E = 320000

NC = 2
NS = 16
NW = NC * NS
EPW = E // NW
CH = 80
GC = 8
NG = 16
EPWP = NG * GC * CH
NACC = 10016
RPT = 624
TAIL0 = RPT * NS
TAILN = N - TAIL0
ZTAILN = NACC - TAIL0

_sc_mesh = plsc.VectorSubcoreMesh(core_axis_name="c", subcore_axis_name="s")


@functools.partial(
    pl.kernel,
    mesh=_sc_mesh,
    out_type=jax.ShapeDtypeStruct((NC, N, D), jnp.float32),
    scratch_types=[
        pltpu.VMEM((GC, CH), jnp.int32),
        pltpu.VMEM((GC, CH), jnp.int32),
        pltpu.VMEM((GC, CH), jnp.int32),
        pltpu.VMEM((GC, CH), jnp.int32),
        pltpu.VMEM((CH, D), jnp.float32),
        pltpu.VMEM((CH, D), jnp.float32),
        pltpu.VMEM((CH, D), jnp.float32),
        pltpu.VMEM((CH, D), jnp.float32),
        pltpu.VMEM_SHARED((NACC, D), jnp.float32),
        pltpu.SemaphoreType.DMA,
        pltpu.SemaphoreType.DMA,
        pltpu.SemaphoreType.DMA,
        pltpu.SemaphoreType.DMA,
        pltpu.SemaphoreType.DMA,
    ],
)
def _segsum_sc(m_hbm, src_hbm, dst_hbm, z_hbm, out_hbm,
               si0, si1, di0, di1, r0buf, r1buf, r2buf, r3buf, acc,
               sem_a, sem_b, sem_c, sem_d, sem_i):
    cid = lax.axis_index("c")
    sid = lax.axis_index("s")
    wid = cid * NS + sid

    row0 = sid * RPT
    pltpu.sync_copy(z_hbm.at[pl.ds(row0, RPT)], acc.at[pl.ds(row0, RPT)])

    @pl.when(sid == NS - 1)
    def _():
        pltpu.sync_copy(z_hbm.at[pl.ds(TAIL0, ZTAILN)],
                        acc.at[pl.ds(TAIL0, ZTAILN)])

    def idx_load_one(arr, g, ref):
        pltpu.async_copy(arr.at[wid, g], ref, sem_i)

    def idx_wait(sref, dref):
        pltpu.make_async_copy(src_hbm.at[wid, 0], sref, sem_i).wait()
        pltpu.make_async_copy(dst_hbm.at[wid, 0], dref, sem_i).wait()

    def gather(sref, k, rbuf, sem):
        pltpu.async_copy(m_hbm.at[sref.at[k]], rbuf, sem)

    def gather_wait(rbuf, sem):
        pltpu.make_async_copy(m_hbm.at[si0.at[0]], rbuf, sem).wait()

    def scatter(rbuf, dref, k):
        pltpu.sync_copy(rbuf, acc.at[dref.at[k]], add=True)

    bufs = [(r0buf, sem_a), (r1buf, sem_b), (r2buf, sem_c), (r3buf, sem_d)]

    pltpu.sync_copy(src_hbm.at[wid, 0], si0)
    pltpu.sync_copy(dst_hbm.at[wid, 0], di0)
    plsc.subcore_barrier()
    idx_load_one(src_hbm, 1, si1)
    idx_load_one(dst_hbm, 1, di1)
    for j in range(4):
        gather(si0, j, bufs[j][0], bufs[j][1])

    def body(g2, _):
        for j in range(8):
            rbuf, sem = bufs[j % 4]
            gather_wait(rbuf, sem)
            if j == 4:
                idx_wait(si1, di1)
            scatter(rbuf, di0, j)
            if j < 4:
                gather(si0, j + 4, rbuf, sem)
            else:
                gather(si1, j - 4, rbuf, sem)

        @pl.when(g2 < NG // 2 - 1)
        def _():
            idx_load_one(src_hbm, 2 * g2 + 2, si0)
            idx_load_one(dst_hbm, 2 * g2 + 2, di0)

        for j in range(8):
            rbuf, sem = bufs[j % 4]
            gather_wait(rbuf, sem)
            scatter(rbuf, di1, j)
            if j < 4:
                gather(si1, j + 4, rbuf, sem)
            else:
                @pl.when(g2 < NG // 2 - 1)
                def _(j=j, rbuf=rbuf, sem=sem):
                    if j == 4:
                        idx_wait(si0, di0)
                    gather(si0, j - 4, rbuf, sem)

        @pl.when(g2 < NG // 2 - 1)
        def _():
            idx_load_one(src_hbm, 2 * g2 + 3, si1)
            idx_load_one(dst_hbm, 2 * g2 + 3, di1)

        return 0

    lax.fori_loop(0, NG // 2, body, 0)

    plsc.subcore_barrier()
    pltpu.sync_copy(acc.at[pl.ds(row0, RPT)],
                    out_hbm.at[cid, pl.ds(row0, RPT)])

    @pl.when(sid == NS - 1)
    def _():
        pltpu.sync_copy(acc.at[pl.ds(TAIL0, TAILN)],
                        out_hbm.at[cid, pl.ds(TAIL0, TAILN)])


_BL = 1000


def _mm(pa, pb, b, W, relu):
    def body(pa_ref, pb_ref, b_ref, w_ref, o_ref):
        s = pa_ref[...] + pb_ref[...]
        y = jnp.dot(s, w_ref[...], preferred_element_type=jnp.float32)
        y = y + b_ref[...]
        o_ref[...] = jnp.maximum(y, 0.0) if relu else y

    return pl.pallas_call(
        body,
        grid=(N // _BL,),
        in_specs=[pl.BlockSpec((_BL, D), lambda i: (i, 0)),
                  pl.BlockSpec((_BL, D), lambda i: (i, 0)),
                  pl.BlockSpec((1, D), lambda i: (0, 0)),
                  pl.BlockSpec((D, D), lambda i: (0, 0))],
        out_specs=pl.BlockSpec((_BL, D), lambda i: (i, 0)),
        out_shape=jax.ShapeDtypeStruct((N, D), jnp.float32),
    )(pa, pb, b, W)


@jax.jit
def kernel(x, adj, W0, b0, W1, b1, W2, b2):
    npad = EPWP - EPW
    pad_src = jnp.broadcast_to(
        (jnp.arange(npad, dtype=jnp.int32) % 64)[None, :], (NW, npad))
    pad_dst = jnp.broadcast_to(
        (N + jnp.arange(npad, dtype=jnp.int32) % (NACC - N))[None, :],
        (NW, npad))
    src2 = jnp.concatenate(
        [adj[0].reshape(NW, EPW), pad_src], axis=1).reshape(NW, NG, GC, CH)
    dst2 = jnp.concatenate(
        [adj[1].reshape(NW, EPW), pad_dst], axis=1).reshape(NW, NG, GC, CH)
    zeros = jnp.zeros((NACC, D), jnp.float32)
    b0r = b0.reshape(1, D)
    b1r = b1.reshape(1, D)
    b2r = b2.reshape(1, D)

    p = _segsum_sc(x, src2, dst2, zeros)
    h = _mm(p[0], p[1], b0r, W0, relu=True)
    p = _segsum_sc(h, src2, dst2, zeros)
    h = _mm(p[0], p[1], b1r, W1, relu=True)
    p = _segsum_sc(h, src2, dst2, zeros)
    return _mm(p[0], p[1], b2r, W2, relu=False)
